# baseline (speedup 1.0000x reference)
import functools

import jax
import jax.numpy as jnp
from jax.experimental import pallas as pl
from jax.experimental.pallas import tpu as pltpu


def _s2d_conv_weight(w_hwio):
    _, _, cin, cout = w_hwio.shape
    w2 = jnp.zeros((3, 3, 2, 2, cin, 2, 2, cout), w_hwio.dtype)
    for a in (0, 1):
        for b in (0, 1):
            for dy in range(3):
                for dx in range(3):
                    t = a - 1 + dy
                    u, r = t // 2, t % 2
                    t2 = b - 1 + dx
                    v, s = t2 // 2, t2 % 2
                    w2 = w2.at[u + 1, v + 1, r, s, :, a, b, :].set(w_hwio[dy, dx])
    return w2.reshape(9 * 4 * cin, 4 * cout)


def _im2col(col_ref, xpad_ref, H, W, Cin):
    for k in range(9):
        dy, dx = divmod(k, 3)
        col_ref[:, k * Cin:(k + 1) * Cin] = (
            xpad_ref[dy:dy + H, dx:dx + W, :].reshape(H * W, Cin))


def _front_kernel(xs_ref, w0_ref, b0_ref, w1_ref, b1_ref, out_ref,
                  xpad0, col0, xpad1, col1, *, Hs, Ws, th):
    r = pl.program_id(1)
    r_last = pl.num_programs(1) - 1
    r0 = r * th

    zc0 = jnp.zeros((th + 4, 1, 12), jnp.bfloat16)
    z2 = jnp.zeros((2, Ws, 12), jnp.bfloat16)
    xpad0[:, 0:1, :] = zc0
    xpad0[:, Ws + 1:Ws + 2, :] = zc0
    xpad0[2:th + 2, 1:Ws + 1, :] = xs_ref[0, pl.ds(r0, th), :, :]

    @pl.when(r == 0)
    def _():
        xpad0[0:2, 1:Ws + 1, :] = z2

    @pl.when(r > 0)
    def _():
        xpad0[0:2, 1:Ws + 1, :] = xs_ref[0, pl.ds(r0 - 2, 2), :, :]

    @pl.when(r == r_last)
    def _():
        xpad0[th + 2:th + 4, 1:Ws + 1, :] = z2

    @pl.when(r < r_last)
    def _():
        xpad0[th + 2:th + 4, 1:Ws + 1, :] = xs_ref[0, pl.ds(r0 + th, 2), :, :]

    _im2col(col0, xpad0, th + 2, Ws, 12)
    acc = jnp.dot(col0[...], w0_ref[...], preferred_element_type=jnp.float32)
    g = acc.reshape(th + 2, Ws, 4, acc.shape[-1] // 4)
    a0 = jnp.maximum(jnp.max(g, axis=2) + b0_ref[...], 0.0).astype(jnp.bfloat16)

    zc1 = jnp.zeros((th + 2, 1, a0.shape[-1]), jnp.bfloat16)
    z1 = jnp.zeros((1, Ws, a0.shape[-1]), jnp.bfloat16)
    xpad1[:, 0:1, :] = zc1
    xpad1[:, Ws + 1:Ws + 2, :] = zc1
    xpad1[0:th + 2, 1:Ws + 1, :] = a0

    @pl.when(r == 0)
    def _():
        xpad1[0:1, 1:Ws + 1, :] = z1

    @pl.when(r == r_last)
    def _():
        xpad1[th + 1:th + 2, 1:Ws + 1, :] = z1

    C1 = w1_ref.shape[1]
    _im2col(col1, xpad1, th, Ws, xpad1.shape[-1])
    acc1 = jnp.dot(col1[...], w1_ref[...], preferred_element_type=jnp.float32)
    y1 = jnp.maximum(acc1 + b1_ref[...], 0.0).reshape(th, Ws, C1)
    p = jnp.max(y1.reshape(th // 2, 2, Ws, C1), axis=1)
    p = jnp.max(p.reshape(th // 2, Ws // 2, 2, C1), axis=2)
    out_ref[0, :, :, :] = p.astype(jnp.bfloat16)


def _tail_kernel(in_ref, *refs, plan):
    n = len(plan)
    w_refs = refs[:n]
    b_refs = refs[n:2 * n]
    tap_refs = refs[2 * n:2 * n + 4]
    it = iter(refs[2 * n + 4:])

    src = in_ref[0, :, :, :]
    tap_i = 0
    for li, (H, W, Cin, Cout, slabs, pool, tap) in enumerate(plan):
        xpad = next(it)
        col = next(it) if H * W >= 4096 else None
        act_ref = next(it) if li + 1 < n else None
        t_ref = tap_refs[tap_i] if tap else None
        if tap:
            tap_i += 1

        zrow = jnp.zeros((1, W + 2, Cin), jnp.bfloat16)
        zcol = jnp.zeros((H, 1, Cin), jnp.bfloat16)
        xpad[0:1, :, :] = zrow
        xpad[H + 1:H + 2, :, :] = zrow
        xpad[1:H + 1, 0:1, :] = zcol
        xpad[1:H + 1, W + 1:W + 2, :] = zcol
        xpad[1:H + 1, 1:W + 1, :] = src

        if col is not None:
            _im2col(col, xpad, H, W, Cin)
        m = H * W // slabs
        rows = H // slabs
        for s in range(slabs):
            if col is not None:
                lhs = col[s * m:(s + 1) * m, :]
            else:
                lhs = jnp.concatenate(
                    [xpad[dy:dy + H, dx:dx + W, :].reshape(H * W, Cin)
                     for dy in range(3) for dx in range(3)], axis=1)
            acc = jnp.dot(lhs, w_refs[li][...], preferred_element_type=jnp.float32)
            y = jnp.maximum(acc + b_refs[li][...], 0.0).reshape(rows, W, Cout)
            yb = y.astype(jnp.bfloat16)
            if t_ref is not None:
                t_ref[0, s * rows:(s + 1) * rows] = yb
            if pool:
                p = jnp.max(y.reshape(rows // 2, 2, W, Cout), axis=1)
                p = jnp.max(p.reshape(rows // 2, W // 2, 2, Cout), axis=2)
                act_ref[s * rows // 2:(s + 1) * rows // 2] = p.astype(jnp.bfloat16)
            elif act_ref is not None:
                act_ref[s * rows:(s + 1) * rows] = yb
        if act_ref is not None:
            src = act_ref[...]


def kernel(x, l0_w, l0_bias, l0_w_hwio, l0_bias_ref,
           l1_w, l1_bias, l1_w_hwio, l1_bias_ref,
           l2_w, l2_bias, l2_w_hwio, l2_bias_ref,
           l3_w, l3_bias, l3_w_hwio, l3_bias_ref,
           l4_w, l4_bias, l4_w_hwio, l4_bias_ref,
           l5_w, l5_bias, l5_w_hwio, l5_bias_ref,
           l6_w, l6_bias, l6_w_hwio, l6_bias_ref,
           l7_w, l7_bias, l7_w_hwio, l7_bias_ref,
           l8_w, l8_bias, l8_w_hwio, l8_bias_ref,
           l9_w, l9_bias, l9_w_hwio, l9_bias_ref,
           last_w, last_bias, last_w_hwio, last_bias_ref):
    N, _, Himg, Wimg = x.shape
    Hs, Ws = Himg // 2, Wimg // 2

    xh = jnp.transpose(x, (0, 2, 3, 1))
    xs = (xh.reshape(N, Hs, 2, Ws, 2, 3)
            .transpose(0, 1, 3, 2, 4, 5)
            .reshape(N, Hs, Ws, 12).astype(jnp.bfloat16))
    w0 = _s2d_conv_weight(l0_w_hwio).astype(jnp.bfloat16)
    b0 = l0_bias_ref.reshape(1, -1)

    th = min(32, Hs)
    n_rb = Hs // th
    C1 = l1_w.shape[1]
    front = pl.pallas_call(
        functools.partial(_front_kernel, Hs=Hs, Ws=Ws, th=th),
        out_shape=jax.ShapeDtypeStruct((N, Hs // 2, Ws // 2, C1), jnp.bfloat16),
        grid=(N, n_rb),
        in_specs=[
            pl.BlockSpec((1, Hs, Ws, 12), lambda b, r: (b, 0, 0, 0)),
            pl.BlockSpec(w0.shape, lambda b, r: (0, 0)),
            pl.BlockSpec(b0.shape, lambda b, r: (0, 0)),
            pl.BlockSpec(l1_w.shape, lambda b, r: (0, 0)),
            pl.BlockSpec(l1_bias.shape, lambda b, r: (0, 0)),
        ],
        out_specs=pl.BlockSpec((1, th // 2, Ws // 2, C1), lambda b, r: (b, r, 0, 0)),
        scratch_shapes=[
            pltpu.VMEM((th + 4, Ws + 2, 12), jnp.bfloat16),
            pltpu.VMEM(((th + 2) * Ws, 108), jnp.bfloat16),
            pltpu.VMEM((th + 2, Ws + 2, 16), jnp.bfloat16),
            pltpu.VMEM((th * Ws, 144), jnp.bfloat16),
        ],
        compiler_params=pltpu.CompilerParams(
            dimension_semantics=("parallel", "parallel"),
            vmem_limit_bytes=int(48 * 2**20)),
    )(xs, w0, b0, l1_w, l1_bias)

    ws = [l2_w, l3_w, l4_w, l5_w, l6_w, l7_w, l8_w, l9_w, last_w]
    bs = [l2_bias, l3_bias, l4_bias, l5_bias, l6_bias, l7_bias,
          l8_bias, l9_bias, last_bias]
    pool_flags = [False, True, False, True, False, True, False, False, False]
    tap_flags = [False, True, False, True, False, True, False, False, True]

    plan = []
    H = W = Hs // 2
    for i in range(9):
        cin = ws[i].shape[0] // 9
        cout = ws[i].shape[1]
        slabs = 2 if H * W >= 4096 else 1
        plan.append((H, W, cin, cout, slabs, pool_flags[i], tap_flags[i]))
        if pool_flags[i]:
            H, W = H // 2, W // 2
    plan = tuple(plan)

    tap_shapes = [(N, p[0], p[1], p[3]) for p in plan if p[6]]
    out_shapes = tuple(jax.ShapeDtypeStruct(s, jnp.bfloat16) for s in tap_shapes)
    out_specs = tuple(pl.BlockSpec((1,) + s[1:], lambda b: (b, 0, 0, 0))
                      for s in tap_shapes)

    scratch = []
    for li, (H, W, cin, cout, slabs, pool, tap) in enumerate(plan):
        scratch.append(pltpu.VMEM((H + 2, W + 2, cin), jnp.bfloat16))
        if H * W >= 4096:
            scratch.append(pltpu.VMEM((H * W, 9 * cin), jnp.bfloat16))
        if li + 1 < len(plan):
            ho, wo = (H // 2, W // 2) if pool else (H, W)
            scratch.append(pltpu.VMEM((ho, wo, cout), jnp.bfloat16))

    in_specs = [pl.BlockSpec((1, Hs // 2, Ws // 2, C1), lambda b: (b, 0, 0, 0))]
    in_specs += [pl.BlockSpec(w.shape, lambda b: (0, 0)) for w in ws]
    in_specs += [pl.BlockSpec(bb.shape, lambda b: (0, 0)) for bb in bs]

    taps = pl.pallas_call(
        functools.partial(_tail_kernel, plan=plan),
        out_shape=out_shapes,
        grid=(N,),
        in_specs=in_specs,
        out_specs=out_specs,
        scratch_shapes=scratch,
        compiler_params=pltpu.CompilerParams(
            dimension_semantics=("parallel",),
            vmem_limit_bytes=int(48 * 2**20)),
    )(front, *ws, *bs)

    t0, t1, t2, t3 = taps
    t3 = t3[..., :last_w_hwio.shape[-1]]
    return [jnp.transpose(t, (0, 3, 1, 2)).astype(jnp.float32)
            for t in (t0, t1, t2, t3)]


# speedup vs baseline: 1.0068x; 1.0068x over previous
import functools

import jax
import jax.numpy as jnp
from jax.experimental import pallas as pl
from jax.experimental.pallas import tpu as pltpu


def _s2d_conv_weight(w_hwio):
    _, _, cin, cout = w_hwio.shape
    w2 = jnp.zeros((3, 3, 2, 2, cin, 2, 2, cout), w_hwio.dtype)
    for a in (0, 1):
        for b in (0, 1):
            for dy in range(3):
                for dx in range(3):
                    t = a - 1 + dy
                    u, r = t // 2, t % 2
                    t2 = b - 1 + dx
                    v, s = t2 // 2, t2 % 2
                    w2 = w2.at[u + 1, v + 1, r, s, :, a, b, :].set(w_hwio[dy, dx])
    return w2.reshape(9, 4 * cin, 4 * cout)


def _zero_borders(xp, H, W, C):
    zrow = jnp.zeros((1, W + 2, C), jnp.bfloat16)
    zcol = jnp.zeros((H, 1, C), jnp.bfloat16)
    xp[0:1, :, :] = zrow
    xp[H + 1:H + 2, :, :] = zrow
    xp[1:H + 1, 0:1, :] = zcol
    xp[1:H + 1, W + 1:W + 2, :] = zcol


def _conv9(src, w_ref, rows, s, W, Cin):
    acc = None
    for k in range(9):
        dy, dx = divmod(k, 3)
        lhs = src[s * rows + dy:s * rows + dy + rows, dx:dx + W, :]
        d = jnp.dot(lhs.reshape(rows * W, Cin), w_ref[k],
                    preferred_element_type=jnp.float32)
        acc = d if acc is None else acc + d
    return acc


def _vgg_kernel(xs_ref, *refs, plan, s2d):
    n = 1 + len(plan)
    w_refs = refs[:n]
    b_refs = refs[n:2 * n]
    tap_refs = refs[2 * n:2 * n + 4]
    xpads = refs[2 * n + 4:]

    Hs, Ws, slabs0 = s2d
    _zero_borders(xpads[0], Hs, Ws, xpads[0].shape[-1])
    for xp, (H, W, Cin, *_rest) in zip(xpads[1:], plan):
        _zero_borders(xp, H, W, Cin)

    xpads[0][1:Hs + 1, 1:Ws + 1, :] = xs_ref[0, :, :, :]
    rows0 = Hs // slabs0
    for s in range(slabs0):
        acc = _conv9(xpads[0], w_refs[0], rows0, s, Ws, 12)
        g = acc.reshape(rows0, Ws, 4, acc.shape[-1] // 4)
        y = jnp.maximum(jnp.max(g, axis=2) + b_refs[0][...], 0.0)
        xpads[1][1 + s * rows0:1 + (s + 1) * rows0, 1:Ws + 1, :] = (
            y.astype(jnp.bfloat16))

    tap_i = 0
    for li, (H, W, Cin, Cout, slabs, pool, tap) in enumerate(plan):
        src = xpads[li + 1]
        dst = xpads[li + 2] if li + 2 < len(xpads) else None
        t_ref = tap_refs[tap_i] if tap else None
        if tap:
            tap_i += 1
        rows = H // slabs
        for s in range(slabs):
            acc = _conv9(src, w_refs[li + 1], rows, s, W, Cin)
            y = jnp.maximum(acc + b_refs[li + 1][...], 0.0).reshape(rows, W, Cout)
            if t_ref is not None:
                t_ref[0, s * rows:(s + 1) * rows] = y.astype(jnp.bfloat16)
            if pool:
                p = jnp.max(y.reshape(rows // 2, 2, W, Cout), axis=1)
                p = jnp.max(p.reshape(rows // 2, W // 2, 2, Cout), axis=2)
                dst[1 + s * rows // 2:1 + (s + 1) * rows // 2, 1:W // 2 + 1, :] = (
                    p.astype(jnp.bfloat16))
            elif dst is not None:
                dst[1 + s * rows:1 + (s + 1) * rows, 1:W + 1, :] = (
                    y.astype(jnp.bfloat16))


def _slabs_for(hw):
    if hw >= 16384:
        return 4
    if hw >= 4096:
        return 2
    return 1


def kernel(x, l0_w, l0_bias, l0_w_hwio, l0_bias_ref,
           l1_w, l1_bias, l1_w_hwio, l1_bias_ref,
           l2_w, l2_bias, l2_w_hwio, l2_bias_ref,
           l3_w, l3_bias, l3_w_hwio, l3_bias_ref,
           l4_w, l4_bias, l4_w_hwio, l4_bias_ref,
           l5_w, l5_bias, l5_w_hwio, l5_bias_ref,
           l6_w, l6_bias, l6_w_hwio, l6_bias_ref,
           l7_w, l7_bias, l7_w_hwio, l7_bias_ref,
           l8_w, l8_bias, l8_w_hwio, l8_bias_ref,
           l9_w, l9_bias, l9_w_hwio, l9_bias_ref,
           last_w, last_bias, last_w_hwio, last_bias_ref):
    N, _, Himg, Wimg = x.shape
    Hs, Ws = Himg // 2, Wimg // 2

    xh = jnp.transpose(x, (0, 2, 3, 1))
    xs = (xh.reshape(N, Hs, 2, Ws, 2, 3)
            .transpose(0, 1, 3, 2, 4, 5)
            .reshape(N, Hs, Ws, 12).astype(jnp.bfloat16))
    w0 = _s2d_conv_weight(l0_w_hwio).astype(jnp.bfloat16)
    b0 = l0_bias_ref.reshape(1, -1)

    ws = [w0] + [w.reshape(9, w.shape[0] // 9, w.shape[1])
                 for w in (l1_w, l2_w, l3_w, l4_w, l5_w, l6_w, l7_w, l8_w,
                           l9_w, last_w)]
    bs = [b0, l1_bias, l2_bias, l3_bias, l4_bias, l5_bias, l6_bias,
          l7_bias, l8_bias, l9_bias, last_bias]

    pool_flags = [True, False, True, False, True, False, True, False, False, False]
    tap_flags = [False, False, True, False, True, False, True, False, False, True]
    plan = []
    H = W = Hs
    for i in range(10):
        cin, cout = ws[i + 1].shape[1], ws[i + 1].shape[2]
        plan.append((H, W, cin, cout, _slabs_for(H * W), pool_flags[i], tap_flags[i]))
        if pool_flags[i]:
            H, W = H // 2, W // 2
    plan = tuple(plan)

    tap_shapes = [(N, p[0], p[1], p[3]) for p in plan if p[6]]
    out_shapes = tuple(jax.ShapeDtypeStruct(s, jnp.bfloat16) for s in tap_shapes)
    out_specs = tuple(pl.BlockSpec((1,) + s[1:], lambda b: (b, 0, 0, 0))
                      for s in tap_shapes)

    scratch = [pltpu.VMEM((Hs + 2, Ws + 2, 12), jnp.bfloat16)]
    scratch += [pltpu.VMEM((p[0] + 2, p[1] + 2, p[2]), jnp.bfloat16) for p in plan]

    in_specs = [pl.BlockSpec((1, Hs, Ws, 12), lambda b: (b, 0, 0, 0))]
    in_specs += [pl.BlockSpec(w.shape, lambda b: (0, 0, 0)) for w in ws]
    in_specs += [pl.BlockSpec(bb.shape, lambda b: (0, 0)) for bb in bs]

    kern = functools.partial(_vgg_kernel, plan=plan,
                             s2d=(Hs, Ws, _slabs_for(Hs * Ws)))
    taps = pl.pallas_call(
        kern,
        out_shape=out_shapes,
        grid=(N,),
        in_specs=in_specs,
        out_specs=out_specs,
        scratch_shapes=scratch,
        compiler_params=pltpu.CompilerParams(
            dimension_semantics=("parallel",),
            vmem_limit_bytes=int(56 * 2**20)),
    )(xs, *ws, *bs)

    t0, t1, t2, t3 = taps
    t3 = t3[..., :last_w_hwio.shape[-1]]
    return [jnp.transpose(t, (0, 3, 1, 2)).astype(jnp.float32)
            for t in (t0, t1, t2, t3)]


# speedup vs baseline: 1.1170x; 1.1095x over previous
import functools

import jax
import jax.numpy as jnp
from jax.experimental import pallas as pl
from jax.experimental.pallas import tpu as pltpu


def _s2d_conv_weight(w_hwio):
    _, _, cin, cout = w_hwio.shape
    w2 = jnp.zeros((3, 3, 2, 2, cin, 2, 2, cout), w_hwio.dtype)
    for a in (0, 1):
        for b in (0, 1):
            for dy in range(3):
                for dx in range(3):
                    t = a - 1 + dy
                    u, r = t // 2, t % 2
                    t2 = b - 1 + dx
                    v, s = t2 // 2, t2 % 2
                    w2 = w2.at[u + 1, v + 1, r, s, :, a, b, :].set(w_hwio[dy, dx])
    return w2.reshape(9, 4 * cin, 4 * cout)


def _init_triple(S1, S0, S2, H, W, C):
    zrow = jnp.zeros((1, W, C), jnp.bfloat16)
    zcol = jnp.zeros((H + 2, 1, C), jnp.bfloat16)
    S1[0:1, :, :] = zrow
    S1[H + 1:H + 2, :, :] = zrow
    S0[:, 0:1, :] = zcol
    S2[:, W - 1:W, :] = zcol


def _shift_copies(S1, S0, S2, W):
    if W > 1:
        S0[:, 1:W, :] = S1[:, 0:W - 1, :]
        S2[:, 0:W - 1, :] = S1[:, 1:W, :]


def _conv9(Sarg, w_ref, rows, s, W, Cin):
    S0, S1, S2 = Sarg
    acc = None
    for k in range(9):
        dy, dx = divmod(k, 3)
        lhs = (S0, S1, S2)[dx][s * rows + dy:s * rows + dy + rows, :, :]
        d = jnp.dot(lhs.reshape(rows * W, Cin), w_ref[k],
                    preferred_element_type=jnp.float32)
        acc = d if acc is None else acc + d
    return acc


def _vgg_kernel(xs_ref, *refs, plan, s2d):
    n = 1 + len(plan)
    w_refs = refs[:n]
    b_refs = refs[n:2 * n]
    tap_refs = refs[2 * n:2 * n + 4]
    xpads = refs[2 * n + 4:]

    Hs, Ws, slabs0 = s2d
    trips = [xpads[3 * i:3 * i + 3] for i in range(len(xpads) // 3)]
    dims = [(Hs, Ws, 12)] + [(p[0], p[1], p[2]) for p in plan]
    for (S1, S0, S2), (H, W, C) in zip(trips, dims):
        _init_triple(S1, S0, S2, H, W, C)

    S1, S0, S2 = trips[0]
    S1[1:Hs + 1, :, :] = xs_ref[0, :, :, :]
    _shift_copies(S1, S0, S2, Ws)
    rows0 = Hs // slabs0
    for s in range(slabs0):
        acc = _conv9((S0, S1, S2), w_refs[0], rows0, s, Ws, 12)
        g = acc.reshape(rows0, Ws, 4, acc.shape[-1] // 4)
        y = jnp.maximum(jnp.max(g, axis=2) + b_refs[0][...], 0.0)
        trips[1][0][1 + s * rows0:1 + (s + 1) * rows0, :, :] = (
            y.astype(jnp.bfloat16))

    tap_i = 0
    for li, (H, W, Cin, Cout, slabs, pool, tap) in enumerate(plan):
        S1, S0, S2 = trips[li + 1]
        _shift_copies(S1, S0, S2, W)
        dst = trips[li + 2][0] if li + 2 < len(trips) else None
        t_ref = tap_refs[tap_i] if tap else None
        if tap:
            tap_i += 1
        rows = H // slabs
        for s in range(slabs):
            acc = _conv9((S0, S1, S2), w_refs[li + 1], rows, s, W, Cin)
            y = jnp.maximum(acc + b_refs[li + 1][...], 0.0).reshape(rows, W, Cout)
            if t_ref is not None:
                t_ref[0, s * rows:(s + 1) * rows] = y.astype(jnp.bfloat16)
            if pool:
                p = jnp.max(y.reshape(rows // 2, 2, W, Cout), axis=1)
                p = jnp.max(p.reshape(rows // 2, W // 2, 2, Cout), axis=2)
                dst[1 + s * rows // 2:1 + (s + 1) * rows // 2, :, :] = (
                    p.astype(jnp.bfloat16))
            elif dst is not None:
                dst[1 + s * rows:1 + (s + 1) * rows, :, :] = (
                    y.astype(jnp.bfloat16))


def _slabs_for(hw):
    if hw >= 16384:
        return 4
    if hw >= 4096:
        return 2
    return 1


def kernel(x, l0_w, l0_bias, l0_w_hwio, l0_bias_ref,
           l1_w, l1_bias, l1_w_hwio, l1_bias_ref,
           l2_w, l2_bias, l2_w_hwio, l2_bias_ref,
           l3_w, l3_bias, l3_w_hwio, l3_bias_ref,
           l4_w, l4_bias, l4_w_hwio, l4_bias_ref,
           l5_w, l5_bias, l5_w_hwio, l5_bias_ref,
           l6_w, l6_bias, l6_w_hwio, l6_bias_ref,
           l7_w, l7_bias, l7_w_hwio, l7_bias_ref,
           l8_w, l8_bias, l8_w_hwio, l8_bias_ref,
           l9_w, l9_bias, l9_w_hwio, l9_bias_ref,
           last_w, last_bias, last_w_hwio, last_bias_ref):
    N, _, Himg, Wimg = x.shape
    Hs, Ws = Himg // 2, Wimg // 2

    xh = jnp.transpose(x, (0, 2, 3, 1))
    xs = (xh.reshape(N, Hs, 2, Ws, 2, 3)
            .transpose(0, 1, 3, 2, 4, 5)
            .reshape(N, Hs, Ws, 12).astype(jnp.bfloat16))
    w0 = _s2d_conv_weight(l0_w_hwio).astype(jnp.bfloat16)
    b0 = l0_bias_ref.reshape(1, -1)

    ws = [w0] + [w.reshape(9, w.shape[0] // 9, w.shape[1])
                 for w in (l1_w, l2_w, l3_w, l4_w, l5_w, l6_w, l7_w, l8_w,
                           l9_w, last_w)]
    bs = [b0, l1_bias, l2_bias, l3_bias, l4_bias, l5_bias, l6_bias,
          l7_bias, l8_bias, l9_bias, last_bias]

    pool_flags = [True, False, True, False, True, False, True, False, False, False]
    tap_flags = [False, False, True, False, True, False, True, False, False, True]
    plan = []
    H = W = Hs
    for i in range(10):
        cin, cout = ws[i + 1].shape[1], ws[i + 1].shape[2]
        plan.append((H, W, cin, cout, _slabs_for(H * W), pool_flags[i], tap_flags[i]))
        if pool_flags[i]:
            H, W = H // 2, W // 2
    plan = tuple(plan)

    tap_shapes = [(N, p[0], p[1], p[3]) for p in plan if p[6]]
    out_shapes = tuple(jax.ShapeDtypeStruct(s, jnp.bfloat16) for s in tap_shapes)
    out_specs = tuple(pl.BlockSpec((1,) + s[1:], lambda b: (b, 0, 0, 0))
                      for s in tap_shapes)

    scratch = [pltpu.VMEM((Hs + 2, Ws, 12), jnp.bfloat16) for _ in range(3)]
    for p in plan:
        scratch += [pltpu.VMEM((p[0] + 2, p[1], p[2]), jnp.bfloat16)
                    for _ in range(3)]

    in_specs = [pl.BlockSpec((1, Hs, Ws, 12), lambda b: (b, 0, 0, 0))]
    in_specs += [pl.BlockSpec(w.shape, lambda b: (0, 0, 0)) for w in ws]
    in_specs += [pl.BlockSpec(bb.shape, lambda b: (0, 0)) for bb in bs]

    kern = functools.partial(_vgg_kernel, plan=plan,
                             s2d=(Hs, Ws, _slabs_for(Hs * Ws)))
    taps = pl.pallas_call(
        kern,
        out_shape=out_shapes,
        grid=(N,),
        in_specs=in_specs,
        out_specs=out_specs,
        scratch_shapes=scratch,
        compiler_params=pltpu.CompilerParams(
            dimension_semantics=("parallel",),
            vmem_limit_bytes=int(56 * 2**20)),
    )(xs, *ws, *bs)

    t0, t1, t2, t3 = taps
    t3 = t3[..., :last_w_hwio.shape[-1]]
    return [jnp.transpose(t, (0, 3, 1, 2)).astype(jnp.float32)
            for t in (t0, t1, t2, t3)]
